# baseline (speedup 1.0000x reference)
import functools

import jax
import jax.numpy as jnp
from jax import lax
from jax.experimental import pallas as pl
from jax.experimental.pallas import tpu as pltpu
from jax.experimental.pallas import tpu_sc as plsc

NUM_Z = 4
Z_VOCAB = 100000
NUM_B = 4
B_VOCAB = 1000
ED = 64
IMG = 512
BATCH = 16384

NC = 2
NS = 16
NW = NC * NS
BPW = BATCH // NW
CH = 128
NCH = BPW // CH
NARM = NUM_Z + NUM_B
PHI_BLKS = IMG // ED


NBUF = 4
DEPTH = 2
NTASK = NARM * NCH


def _body(idx_hbm, phi_hbm, zf_hbm, bf_hbm, out_hbm, idx_v, rows_v, gsem, wsem, psem):
    wid = lax.axis_index("s") * NC + lax.axis_index("c")
    base = wid * BPW

    pltpu.sync_copy(idx_hbm.at[wid], idx_v)

    phi_d = pltpu.async_copy(
        phi_hbm.at[pl.ds(base, BPW)],
        out_hbm.at[pl.ds(base, BPW), pl.ds(NARM, PHI_BLKS)],
        psem,
    )

    def gather(t, s):
        a, c = divmod(t, NCH)
        table = zf_hbm if a < NUM_Z else bf_hbm
        return pltpu.async_copy(table.at[idx_v.at[a, c]], rows_v.at[s], gsem.at[s])

    def write(t, s):
        a, c = divmod(t, NCH)
        return pltpu.async_copy(
            rows_v.at[s], out_hbm.at[pl.ds(base + c * CH, CH), a], wsem.at[s]
        )

    gd = [None] * NBUF
    wd = [None] * NBUF
    for t in range(DEPTH):
        gd[t % NBUF] = gather(t, t % NBUF)
    for t in range(NTASK):
        s = t % NBUF
        gd[s].wait()
        wd[s] = write(t, s)
        tn = t + DEPTH
        if tn < NTASK:
            sn = tn % NBUF
            if wd[sn] is not None:
                wd[sn].wait()
            gd[sn] = gather(tn, sn)
    for s in range(NBUF):
        if wd[s] is not None:
            wd[s].wait()
    phi_d.wait()


@jax.jit
def _run(idx, phi_r, z_flat, b_flat):
    mesh = plsc.VectorSubcoreMesh(
        core_axis_name="c", subcore_axis_name="s", num_cores=NC, num_subcores=NS
    )
    return pl.kernel(
        _body,
        out_type=jax.ShapeDtypeStruct((BATCH, NARM + PHI_BLKS, ED), jnp.float32),
        mesh=mesh,
        scratch_types=[
            pltpu.VMEM((NARM, NCH, CH), jnp.int32),
            pltpu.VMEM((NBUF, CH, ED), jnp.float32),
            pltpu.SemaphoreType.DMA((NBUF,)),
            pltpu.SemaphoreType.DMA((NBUF,)),
            pltpu.SemaphoreType.DMA,
        ],
        compiler_params=pltpu.CompilerParams(use_tc_tiling_on_sc=False),
    )(idx, phi_r, z_flat, b_flat)


def kernel(z, beta, phi_x, Z_tables, B_tables):
    zoff = jnp.arange(NUM_Z, dtype=jnp.int32) * Z_VOCAB
    boff = jnp.arange(NUM_B, dtype=jnp.int32) * B_VOCAB
    zi = (z.astype(jnp.int32) + zoff[None, :]).T
    bi = (beta.astype(jnp.int32) + boff[None, :]).T
    idx = jnp.concatenate([zi, bi], axis=0)
    idx = (
        idx.reshape(NARM, NW, NCH * CH).transpose(1, 0, 2).reshape(NW, NARM, NCH, CH)
    )
    out = _run(
        idx,
        phi_x.reshape(BATCH, PHI_BLKS, ED),
        Z_tables.reshape(NUM_Z * Z_VOCAB, ED),
        B_tables.reshape(NUM_B * B_VOCAB, ED),
    )
    return out.reshape(BATCH, (NARM + PHI_BLKS) * ED)


# speedup vs baseline: 2.5981x; 2.5981x over previous
import functools

import jax
import jax.numpy as jnp
from jax import lax
from jax.experimental import pallas as pl
from jax.experimental.pallas import tpu as pltpu
from jax.experimental.pallas import tpu_sc as plsc

NUM_Z = 4
Z_VOCAB = 100000
NUM_B = 4
B_VOCAB = 1000
ED = 64
IMG = 512
BATCH = 16384

NC = 2
NS = 16
NW = NC * NS
BPW = BATCH // NW
CH = 128
NCH = BPW // CH
NARM = NUM_Z + NUM_B
PHI_BLKS = IMG // ED


NBUF = 4
DEPTH = 2
NTASK = NARM * NCH
PHI_CHUNKS = 8


def _body(idx_hbm, phi_hbm, zf_hbm, bf_hbm, out_hbm, idx_v, rows_v, phi_v, gsem, wsem, psem):
    wid = lax.axis_index("s") * NC + lax.axis_index("c")
    base = wid * BPW

    pltpu.sync_copy(idx_hbm.at[wid], idx_v)

    def gather(t, s):
        a, c = divmod(t, NCH)
        table = zf_hbm if a < NUM_Z else bf_hbm
        return pltpu.async_copy(table.at[idx_v.at[a, c]], rows_v.at[s], gsem.at[s])

    def write(t, s):
        a, c = divmod(t, NCH)
        return pltpu.async_copy(
            rows_v.at[s], out_hbm.at[pl.ds(base + c * CH, CH), a], wsem.at[s]
        )

    gd = [None] * NBUF
    wd = [None] * NBUF
    for t in range(DEPTH):
        gd[t % NBUF] = gather(t, t % NBUF)
    for t in range(NTASK):
        s = t % NBUF
        gd[s].wait()
        wd[s] = write(t, s)
        tn = t + DEPTH
        if tn < NTASK:
            sn = tn % NBUF
            if wd[sn] is not None:
                wd[sn].wait()
            gd[sn] = gather(tn, sn)
    for s in range(NBUF):
        if wd[s] is not None:
            wd[s].wait()

    PCH = BPW // PHI_CHUNKS
    pin = [None, None]
    pout = [None, None]
    for c in range(PHI_CHUNKS):
        s = c % 2
        if pout[s] is not None:
            pout[s].wait()
        pin[s] = pltpu.async_copy(
            phi_hbm.at[pl.ds(base + c * PCH, PCH)], phi_v.at[s], psem.at[s]
        )
        pin[s].wait()
        pout[s] = pltpu.async_copy(
            phi_v.at[s],
            out_hbm.at[pl.ds(base + c * PCH, PCH), pl.ds(NARM, PHI_BLKS)],
            wsem.at[s],
        )
    for s in range(2):
        if pout[s] is not None:
            pout[s].wait()


@jax.jit
def _run(idx, phi_r, z_flat, b_flat):
    mesh = plsc.VectorSubcoreMesh(
        core_axis_name="c", subcore_axis_name="s", num_cores=NC, num_subcores=NS
    )
    return pl.kernel(
        _body,
        out_type=jax.ShapeDtypeStruct((BATCH, NARM + PHI_BLKS, ED), jnp.float32),
        mesh=mesh,
        scratch_types=[
            pltpu.VMEM((NARM, NCH, CH), jnp.int32),
            pltpu.VMEM((NBUF, CH, ED), jnp.float32),
            pltpu.VMEM((2, BPW // PHI_CHUNKS, PHI_BLKS, ED), jnp.float32),
            pltpu.SemaphoreType.DMA((NBUF,)),
            pltpu.SemaphoreType.DMA((NBUF,)),
            pltpu.SemaphoreType.DMA((2,)),
        ],
        compiler_params=pltpu.CompilerParams(use_tc_tiling_on_sc=False),
    )(idx, phi_r, z_flat, b_flat)


def kernel(z, beta, phi_x, Z_tables, B_tables):
    zoff = jnp.arange(NUM_Z, dtype=jnp.int32) * Z_VOCAB
    boff = jnp.arange(NUM_B, dtype=jnp.int32) * B_VOCAB
    zi = (z.astype(jnp.int32) + zoff[None, :]).T
    bi = (beta.astype(jnp.int32) + boff[None, :]).T
    idx = jnp.concatenate([zi, bi], axis=0)
    idx = (
        idx.reshape(NARM, NW, NCH * CH).transpose(1, 0, 2).reshape(NW, NARM, NCH, CH)
    )
    out = _run(
        idx,
        phi_x.reshape(BATCH, PHI_BLKS, ED),
        Z_tables.reshape(NUM_Z * Z_VOCAB, ED),
        B_tables.reshape(NUM_B * B_VOCAB, ED),
    )
    return out.reshape(BATCH, (NARM + PHI_BLKS) * ED)


# speedup vs baseline: 4.3618x; 1.6788x over previous
import functools

import jax
import jax.numpy as jnp
from jax import lax
from jax.experimental import pallas as pl
from jax.experimental.pallas import tpu as pltpu
from jax.experimental.pallas import tpu_sc as plsc

NUM_Z = 4
Z_VOCAB = 100000
NUM_B = 4
B_VOCAB = 1000
ED = 64
BATCH = 16384
EMB_D = (NUM_Z + NUM_B) * ED

NC = 2
NS = 16
NW = NC * NS
BPW = BATCH // NW
CH = 128
NCH = BPW // CH
NARM = NUM_Z + NUM_B

NBUF = 4
DEPTH = 3
NTASK = NARM * NCH


def _body(idx_hbm, zf_hbm, bf_hbm, out_hbm, idx_v, rows_v, gsem, wsem):
    wid = lax.axis_index("s") * NC + lax.axis_index("c")
    base = wid * BPW

    pltpu.sync_copy(idx_hbm.at[wid], idx_v)

    def gather(t, s):
        a, c = divmod(t, NCH)
        table = zf_hbm if a < NUM_Z else bf_hbm
        return pltpu.async_copy(table.at[idx_v.at[c, a]], rows_v.at[s], gsem.at[s])

    def write(t, s):
        a, c = divmod(t, NCH)
        return pltpu.async_copy(
            rows_v.at[s],
            out_hbm.at[pl.ds(base + c * CH, CH), pl.ds(a * ED, ED)],
            wsem.at[s],
        )

    gd = [None] * NBUF
    wd = [None] * NBUF
    for t in range(DEPTH):
        gd[t % NBUF] = gather(t, t % NBUF)
    for t in range(NTASK):
        s = t % NBUF
        gd[s].wait()
        wd[s] = write(t, s)
        tn = t + DEPTH
        if tn < NTASK:
            sn = tn % NBUF
            if wd[sn] is not None:
                wd[sn].wait()
            gd[sn] = gather(tn, sn)
    for s in range(NBUF):
        if wd[s] is not None:
            wd[s].wait()


@jax.jit
def _run(idx, z_flat, b_flat):
    mesh = plsc.VectorSubcoreMesh(
        core_axis_name="c", subcore_axis_name="s", num_cores=NC, num_subcores=NS
    )
    return pl.kernel(
        _body,
        out_type=jax.ShapeDtypeStruct((BATCH, EMB_D), jnp.float32),
        mesh=mesh,
        scratch_types=[
            pltpu.VMEM((NCH, NARM, CH), jnp.int32),
            pltpu.VMEM((NBUF, CH, ED), jnp.float32),
            pltpu.SemaphoreType.DMA((NBUF,)),
            pltpu.SemaphoreType.DMA((NBUF,)),
        ],
        compiler_params=pltpu.CompilerParams(use_tc_tiling_on_sc=False),
    )(idx, z_flat, b_flat)


def kernel(z, beta, phi_x, Z_tables, B_tables):
    zoff = jnp.arange(NUM_Z, dtype=jnp.int32) * Z_VOCAB
    boff = jnp.arange(NUM_B, dtype=jnp.int32) * B_VOCAB
    zi = (z.astype(jnp.int32) + zoff[None, :]).T
    bi = (beta.astype(jnp.int32) + boff[None, :]).T
    idx8 = jnp.concatenate([zi, bi], axis=0)
    idx = (
        idx8.reshape(NARM, BATCH // CH, CH)
        .transpose(1, 0, 2)
        .reshape(NW, NCH, NARM, CH)
    )
    emb = _run(
        idx,
        Z_tables.reshape(NUM_Z * Z_VOCAB, ED),
        B_tables.reshape(NUM_B * B_VOCAB, ED),
    )
    return jnp.concatenate([emb, phi_x], axis=1)
